# baseline (speedup 1.0000x reference)
import functools

import jax
import jax.numpy as jnp
from jax import lax
from jax.experimental import pallas as pl
from jax.experimental.pallas import tpu as pltpu
from jax.experimental.pallas import tpu_sc as plsc

D_MODEL = 128
G = 128
NB = 2


@functools.cache
def _make_gather(B: int):
    info = plsc.get_sparse_core_info()
    nc, ns = info.num_cores, info.num_subcores
    nw = nc * ns
    b_per_w = B // nw
    n_g = b_per_w // G
    assert b_per_w * nw == B and n_g * G == b_per_w and n_g % NB == 0

    n_rounds = n_g // NB
    assert n_rounds % 2 == 0 and n_rounds >= 6
    mesh = plsc.VectorSubcoreMesh(core_axis_name="c", subcore_axis_name="s")
    scratch = [pltpu.VMEM((n_g, G), jnp.int32)]
    scratch += [pltpu.VMEM((G, D_MODEL), jnp.float32) for _ in range(2 * NB)]
    scratch += [pltpu.SemaphoreType.DMA for _ in range(4 * NB)]

    @functools.partial(
        pl.kernel,
        out_type=jax.ShapeDtypeStruct((B, D_MODEL), jnp.float32),
        mesh=mesh,
        scratch_types=scratch,
    )
    def k(table_hbm, idx_hbm, out_hbm, idx_v, *bufs_sems):
        bufs = bufs_sems[: 2 * NB]
        gsems = bufs_sems[2 * NB : 4 * NB]
        osems = bufs_sems[4 * NB :]
        halves = (tuple(range(NB)), tuple(range(NB, 2 * NB)))
        wid = lax.axis_index("s") * nc + lax.axis_index("c")
        base = wid * b_per_w
        pltpu.sync_copy(idx_hbm.at[wid], idx_v)

        def gather(g, s):
            pltpu.async_copy(table_hbm.at[idx_v.at[g]], bufs[s], gsems[s])

        def wait_gather(s):
            pltpu.make_async_copy(table_hbm.at[pl.ds(0, G)], bufs[s], gsems[s]).wait()

        def out_start(g, s):
            pltpu.async_copy(bufs[s], out_hbm.at[pl.ds(base + g * G, G)], osems[s])

        def wait_out(s):
            pltpu.make_async_copy(bufs[s], out_hbm.at[pl.ds(base, G)], osems[s]).wait()

        def round_body(r, parity, fire_next=True, wait_oth=True):
            cur = halves[parity]
            oth = halves[1 - parity]
            for i in range(NB):
                wait_gather(cur[i])
                out_start(r * NB + i, cur[i])
            for i in range(NB):
                if wait_oth:
                    wait_out(oth[i])
                if fire_next:
                    gather((r + 1) * NB + i, oth[i])

        for i in range(NB):
            gather(i, halves[0][i])
        round_body(0, 0, fire_next=True, wait_oth=False)
        round_body(1, 1)

        def dbl(i, carry):
            r = 2 + 2 * i
            round_body(r, 0)
            round_body(r + 1, 1)
            return carry

        lax.fori_loop(0, (n_rounds - 4) // 2, dbl, 0)

        round_body(n_rounds - 2, 0)
        round_body(n_rounds - 1, 1, fire_next=False)
        for i in range(NB):
            wait_out(halves[1][i])

    return k


def kernel(table, event):
    bsz, seq = event.shape
    B = bsz * seq
    idx = event.reshape(-1).astype(jnp.int32)
    info = plsc.get_sparse_core_info()
    nw = info.num_cores * info.num_subcores
    idx3 = idx.reshape(nw, B // nw // G, G)
    out = _make_gather(B)(table, idx3)
    tc = _tc_copy(table)
    out = out.at[0, 0].add(tc[0, 0])
    return out.reshape(bsz, seq, D_MODEL)


def _tc_copy(table):
    n_rows = 98304
    blk = 1024

    def body(t_ref, o_ref):
        o_ref[...] = t_ref[...] + 1.0

    return pl.pallas_call(
        body,
        grid=(8, n_rows // blk),
        in_specs=[pl.BlockSpec((blk, 128), lambda i, j: (j, 0))],
        out_specs=pl.BlockSpec((blk, 128), lambda i, j: (j, 0)),
        out_shape=jax.ShapeDtypeStruct((n_rows, 128), jnp.float32),
    )(table[:n_rows])


# speedup vs baseline: 2.5247x; 2.5247x over previous
import functools

import jax
import jax.numpy as jnp
from jax import lax
from jax.experimental import pallas as pl
from jax.experimental.pallas import tpu as pltpu
from jax.experimental.pallas import tpu_sc as plsc

D_MODEL = 128
G = 128
NB = 2


@functools.cache
def _make_gather(B: int):
    info = plsc.get_sparse_core_info()
    nc, ns = info.num_cores, info.num_subcores
    nw = nc * ns
    b_per_w = B // nw
    n_g = b_per_w // G
    assert b_per_w * nw == B and n_g * G == b_per_w and n_g % NB == 0

    n_rounds = n_g // NB
    assert n_rounds % 2 == 0 and n_rounds >= 6
    mesh = plsc.VectorSubcoreMesh(core_axis_name="c", subcore_axis_name="s")
    scratch = [pltpu.VMEM((n_g, G), jnp.int32)]
    scratch += [pltpu.VMEM((NB * G, D_MODEL), jnp.float32) for _ in range(2)]
    scratch += [pltpu.SemaphoreType.DMA for _ in range(2 * NB + 2)]

    @functools.partial(
        pl.kernel,
        out_type=jax.ShapeDtypeStruct((B, D_MODEL), jnp.float32),
        mesh=mesh,
        scratch_types=scratch,
    )
    def k(table_hbm, idx_hbm, out_hbm, idx_v, *bufs_sems):
        bufs = bufs_sems[:2]
        gsems = bufs_sems[2 : 2 + 2 * NB]
        osems = bufs_sems[2 + 2 * NB :]
        wid = lax.axis_index("s") * nc + lax.axis_index("c")
        base = wid * b_per_w
        pltpu.sync_copy(idx_hbm.at[wid], idx_v)

        def gather(g, parity, i):
            pltpu.async_copy(
                table_hbm.at[idx_v.at[g]],
                bufs[parity].at[pl.ds(i * G, G)],
                gsems[parity * NB + i],
            )

        def wait_gather(parity, i):
            pltpu.make_async_copy(
                table_hbm.at[pl.ds(0, G)],
                bufs[parity].at[pl.ds(i * G, G)],
                gsems[parity * NB + i],
            ).wait()

        def out_start(r, parity):
            pltpu.async_copy(
                bufs[parity], out_hbm.at[pl.ds(base + r * NB * G, NB * G)], osems[parity]
            )

        def wait_out(parity):
            pltpu.make_async_copy(
                bufs[parity], out_hbm.at[pl.ds(base, NB * G)], osems[parity]
            ).wait()

        def round_body(r, parity, fire_next=True, wait_oth=True):
            for i in range(NB):
                wait_gather(parity, i)
            out_start(r, parity)
            if wait_oth:
                wait_out(1 - parity)
            if fire_next:
                for i in range(NB):
                    gather((r + 1) * NB + i, 1 - parity, i)

        for i in range(NB):
            gather(i, 0, i)
        round_body(0, 0, fire_next=True, wait_oth=False)
        round_body(1, 1)

        def dbl(i, carry):
            r = 2 + 2 * i
            round_body(r, 0)
            round_body(r + 1, 1)
            return carry

        lax.fori_loop(0, (n_rounds - 4) // 2, dbl, 0)

        round_body(n_rounds - 2, 0)
        round_body(n_rounds - 1, 1, fire_next=False)
        wait_out(1)

    return k


def kernel(table, event):
    bsz, seq = event.shape
    B = bsz * seq
    idx = event.reshape(-1).astype(jnp.int32)
    info = plsc.get_sparse_core_info()
    nw = info.num_cores * info.num_subcores
    idx3 = idx.reshape(nw, B // nw // G, G)
    out = _make_gather(B)(table, idx3)
    return out.reshape(bsz, seq, D_MODEL)
